# Initial kernel scaffold; baseline (speedup 1.0000x reference)
import jax
import jax.numpy as jnp
from jax.experimental import pallas as pl


def kernel(x, edge_weight, A2, W1, b1, W2, b2, W3, b3, edge_index):
    raise NotImplementedError("write your pallas kernel here")


# speedup vs baseline: 4.9046x; 4.9046x over previous
import functools

import jax
import jax.numpy as jnp
from jax import lax
from jax.experimental import pallas as pl
from jax.experimental.pallas import tpu as pltpu
from jax.experimental.pallas import tpu_sc as plsc

N = 10000
E = 320000
D = 128
HALF = 64
HOP = 10

NS = 16
NC = 2
C = 128
CHUNKS = 158
EPT = CHUNKS * C
NPT = N // NS


def _mlp1_body(x_ref, w1t_ref, b1_ref, w2t_ref, b2_ref, a2_ref, x2s_ref, rs_ref):
    h = jnp.dot(x_ref[...], w1t_ref[...], preferred_element_type=jnp.float32)
    h = jnp.maximum(h + b1_ref[...], 0.0)
    x2 = jnp.dot(h, w2t_ref[...], preferred_element_type=jnp.float32) + b2_ref[...]
    r = a2_ref[...] * x2
    x2s_ref[0] = x2[:, :HALF]
    x2s_ref[1] = x2[:, HALF:]
    rs_ref[0] = r[:, :HALF]
    rs_ref[1] = r[:, HALF:]


def _mlp2_body(hs_ref, w3t_ref, b3_ref, out_ref):
    h = jnp.concatenate([hs_ref[0], hs_ref[1]], axis=-1)
    h = jnp.maximum(h, 0.0)
    out_ref[...] = jnp.dot(h, w3t_ref[...], preferred_element_type=jnp.float32) + b3_ref[...]


def _sc_body(x2_hbm, r_hbm, e3_hbm, out_hbm,
             buf_a, buf_b, eb0, eb1, rows_v, sem0, sem1):
    cid = lax.axis_index("c")
    sid = lax.axis_index("s")
    node_lo = sid * NPT

    pltpu.sync_copy(x2_hbm.at[cid, pl.ds(node_lo, NPT)],
                    buf_a.at[pl.ds(node_lo, NPT)])

    def process(eb, src, dst):
        pltpu.sync_copy(src.at[eb.at[0]], rows_v)

        @pl.loop(0, C)
        def _edge(e):
            wi = plsc.load_gather(
                eb, [jnp.full((16,), 2, jnp.int32), jnp.full((16,), e, jnp.int32)])
            wv = plsc.bitcast(wi, jnp.float32)
            for d in range(HALF // 16):
                sl = pl.ds(d * 16, 16)
                rows_v[e, sl] = rows_v[e, sl] * wv

        pltpu.sync_copy(rows_v, dst.at[eb.at[1]], add=True)

    for k in range(HOP):
        src, dst = (buf_a, buf_b) if k % 2 == 0 else (buf_b, buf_a)

        pltpu.sync_copy(r_hbm.at[cid, pl.ds(node_lo, NPT)],
                        dst.at[pl.ds(node_lo, NPT)])
        plsc.subcore_barrier()

        pltpu.async_copy(e3_hbm.at[sid, 0], eb0, sem0)

        @pl.loop(0, CHUNKS // 2)
        def _chunk(jj):
            j0 = 2 * jj
            pltpu.async_copy(e3_hbm.at[sid, j0 + 1], eb1, sem1)
            pltpu.make_async_copy(e3_hbm.at[sid, j0], eb0, sem0).wait()
            process(eb0, src, dst)

            @pl.when(j0 + 2 < CHUNKS)
            def _():
                pltpu.async_copy(e3_hbm.at[sid, j0 + 2], eb0, sem0)

            pltpu.make_async_copy(e3_hbm.at[sid, j0 + 1], eb1, sem1).wait()
            process(eb1, src, dst)

        plsc.subcore_barrier()

    pltpu.sync_copy(buf_a.at[pl.ds(node_lo, NPT)],
                    out_hbm.at[cid, pl.ds(node_lo, NPT)])


@jax.jit
def kernel(x, edge_weight, A2, W1, b1, W2, b2, W3, b3, edge_index):
    row = edge_index[0]
    col = edge_index[1]
    pad = NS * EPT - E
    row_p = jnp.concatenate([row, jnp.zeros((pad,), jnp.int32)]).reshape(NS, CHUNKS, 1, C)
    col_p = jnp.concatenate([col, jnp.zeros((pad,), jnp.int32)]).reshape(NS, CHUNKS, 1, C)
    w_p = jax.lax.bitcast_convert_type(
        jnp.concatenate([edge_weight, jnp.zeros((pad,), jnp.float32)]), jnp.int32
    ).reshape(NS, CHUNKS, 1, C)
    e3 = jnp.concatenate([col_p, row_p, w_p], axis=2)

    x2s, rs = pl.pallas_call(
        _mlp1_body,
        out_shape=(
            jax.ShapeDtypeStruct((NC, N, HALF), jnp.float32),
            jax.ShapeDtypeStruct((NC, N, HALF), jnp.float32),
        ),
    )(x, W1.T, b1.reshape(1, D), W2.T, b2.reshape(1, D), A2)

    sc_fn = pl.kernel(
        _sc_body,
        out_type=jax.ShapeDtypeStruct((NC, N, HALF), jnp.float32),
        mesh=plsc.VectorSubcoreMesh(core_axis_name="c", subcore_axis_name="s"),
        compiler_params=pltpu.CompilerParams(
            use_tc_tiling_on_sc=False, needs_layout_passes=False),
        scratch_types=[
            pltpu.VMEM_SHARED((N, HALF), jnp.float32),
            pltpu.VMEM_SHARED((N, HALF), jnp.float32),
            pltpu.VMEM((3, C), jnp.int32),
            pltpu.VMEM((3, C), jnp.int32),
            pltpu.VMEM((C, HALF), jnp.float32),
            pltpu.SemaphoreType.DMA,
            pltpu.SemaphoreType.DMA,
        ],
    )
    hs = sc_fn(x2s, rs, e3)

    out = pl.pallas_call(
        _mlp2_body,
        out_shape=jax.ShapeDtypeStruct((N, D), jnp.float32),
    )(hs, W3.T, b3.reshape(1, D))
    return out
